# baseline (speedup 1.0000x reference)
import functools

import jax
import jax.numpy as jnp
from jax.experimental import pallas as pl
from jax.experimental.pallas import tpu as pltpu

_VMEM_LIMIT = 57 * 1024 * 1024


def _mm_t_bias_relu_kernel(x_ref, w_ref, b_ref, o_ref):
    acc = jax.lax.dot_general(x_ref[...], w_ref[...],
                              (((0,), (0,)), ((), ())),
                              preferred_element_type=jnp.float32)
    acc = jnp.maximum(acc + b_ref[...], 0.0)
    o_ref[...] = acc.astype(o_ref.dtype)


def _conv3x3_kernel(x_ref, w_ref, b_ref, o_ref, xp_s, *, th, w_out, cin,
                    nchw_out=False):
    h = pl.program_id(1)
    hh = x_ref.shape[1]

    @pl.when(h == 0)
    def _():
        for r in range(0, hh, 64):
            rr = min(64, hh - r)
            xp_s[1 + r:1 + r + rr, 1:w_out + 1, :] = x_ref[0, r:r + rr, :, :]
        xp_s[0:1, 1:w_out + 1, :] = x_ref[0, 1:2, :, :]
        xp_s[hh + 1:hh + 2, 1:w_out + 1, :] = x_ref[0, hh - 2:hh - 1, :, :]
        xp_s[:, 0:1, :] = xp_s[:, 2:3, :]
        xp_s[:, w_out + 1:w_out + 2, :] = xp_s[:, w_out - 1:w_out, :]

    rows = xp_s[pl.ds(h * th, th + 2), :, :]
    pieces = [rows[di:di + th, dj:dj + w_out, :]
              for di in range(3) for dj in range(3)]
    patches = jnp.concatenate(pieces, axis=-1)
    p2 = patches.reshape(th * w_out, 9 * cin)
    acc = jnp.dot(p2, w_ref[...], preferred_element_type=jnp.float32)
    acc = jnp.maximum(acc + b_ref[...], 0.0)
    if nchw_out:
        o_ref[0, ...] = acc.T.reshape(-1, th, w_out).astype(o_ref.dtype)
    else:
        o_ref[0, ...] = acc.reshape(th, w_out, -1).astype(o_ref.dtype)


def _bf16_lo(q):
    return pltpu.bitcast(q << 16, jnp.float32)


def _bf16_hi(q):
    return pltpu.bitcast(q & jnp.int32(-65536), jnp.float32)


def _argmax4(v0, v1, v2, v3, w_in):
    best = v0
    delta = jnp.zeros(best.shape, jnp.int32)
    for v, d in ((v1, 1), (v2, w_in), (v3, w_in + 1)):
        upd = v > best
        best = jnp.where(upd, v, best)
        delta = jnp.where(upd, jnp.int32(d), delta)
    return best, delta


def _conv3x3_pool_kernel(x_ref, w_ref, b_ref, o_ref, i_ref, xp_s,
                         *, th, w_out, cin):
    h = pl.program_id(1)
    hh = x_ref.shape[1]

    @pl.when(h == 0)
    def _():
        for r in range(0, hh, 64):
            rr = min(64, hh - r)
            xp_s[1 + r:1 + r + rr, 1:w_out + 1, :] = x_ref[0, r:r + rr, :, :]
        xp_s[0:1, 1:w_out + 1, :] = x_ref[0, 1:2, :, :]
        xp_s[hh + 1:hh + 2, 1:w_out + 1, :] = x_ref[0, hh - 2:hh - 1, :, :]
        xp_s[:, 0:1, :] = xp_s[:, 2:3, :]
        xp_s[:, w_out + 1:w_out + 2, :] = xp_s[:, w_out - 1:w_out, :]

    rows = xp_s[pl.ds(h * th, th + 2), :, :]
    pieces = [rows[di:di + th, dj:dj + w_out, :]
              for di in range(3) for dj in range(3)]
    patches = jnp.concatenate(pieces, axis=-1)
    p2 = patches.reshape(th * w_out, 9 * cin)
    acc = jnp.dot(p2, w_ref[...], preferred_element_type=jnp.float32)
    acc = jnp.maximum(acc + b_ref[...], 0.0)
    cout = w_ref.shape[1]
    y = acc.astype(jnp.bfloat16).reshape(th, w_out, cout)
    zi = pltpu.bitcast(y, jnp.int32)
    z = zi.reshape(th // 2, 2, w_out // 2, cout)
    r0, r1 = z[:, 0], z[:, 1]
    best, delta = _argmax4(_bf16_lo(r0), _bf16_hi(r0),
                           _bf16_lo(r1), _bf16_hi(r1), w_out)
    hrow = h * (th // 2) + jax.lax.broadcasted_iota(jnp.int32, best.shape, 0)
    ww = jax.lax.broadcasted_iota(jnp.int32, best.shape, 1)
    o_ref[0, ...] = best.astype(o_ref.dtype)
    idxv = 2 * hrow * w_out + 2 * ww + delta
    th2, wo2 = th // 2, w_out // 2
    idx_t = idxv.reshape(th2 * wo2, cout).T
    i_ref[0, ...] = idx_t.reshape(cout, th2, wo2)


def _conv3x3(x, w, b, th_pref):
    n, h, wd, c = x.shape
    cout = w.shape[1]
    th = min(th_pref, h)
    nchw_out = cout == 512
    out_dtype = jnp.float32 if nchw_out else jnp.bfloat16
    out_shape = (n, cout, h, wd) if nchw_out else (n, h, wd, cout)
    if nchw_out:
        out_spec = pl.BlockSpec((1, cout, th, wd), lambda i, j: (i, 0, j, 0))
    else:
        out_spec = pl.BlockSpec((1, th, wd, cout), lambda i, j: (i, j, 0, 0))
    return pl.pallas_call(
        functools.partial(_conv3x3_kernel, th=th, w_out=wd, cin=c,
                          nchw_out=nchw_out),
        out_shape=jax.ShapeDtypeStruct(out_shape, out_dtype),
        grid=(n, h // th),
        in_specs=[
            pl.BlockSpec((1, h, wd, c), lambda i, j: (i, 0, 0, 0)),
            pl.BlockSpec((9 * c, cout), lambda i, j: (0, 0)),
            pl.BlockSpec((1, cout), lambda i, j: (0, 0)),
        ],
        out_specs=out_spec,
        scratch_shapes=[pltpu.VMEM((h + 2, wd + 2, c), jnp.bfloat16)],
        compiler_params=pltpu.CompilerParams(
            dimension_semantics=("parallel", "arbitrary"),
            vmem_limit_bytes=_VMEM_LIMIT),
    )(x, w, b)


def _conv3x3_pool(x, w, b, th_pref):
    n, h, wd, c = x.shape
    cout = w.shape[1]
    th = min(th_pref, h)
    return pl.pallas_call(
        functools.partial(_conv3x3_pool_kernel, th=th, w_out=wd, cin=c),
        out_shape=(
            jax.ShapeDtypeStruct((n, h // 2, wd // 2, cout), jnp.bfloat16),
            jax.ShapeDtypeStruct((n, cout, h // 2, wd // 2), jnp.int32)),
        grid=(n, h // th),
        in_specs=[
            pl.BlockSpec((1, h, wd, c), lambda i, j: (i, 0, 0, 0)),
            pl.BlockSpec((9 * c, cout), lambda i, j: (0, 0)),
            pl.BlockSpec((1, cout), lambda i, j: (0, 0)),
        ],
        out_specs=(
            pl.BlockSpec((1, th // 2, wd // 2, cout),
                         lambda i, j: (i, j, 0, 0)),
            pl.BlockSpec((1, cout, th // 2, wd // 2),
                         lambda i, j: (i, 0, j, 0))),
        scratch_shapes=[pltpu.VMEM((h + 2, wd + 2, c), jnp.bfloat16)],
        compiler_params=pltpu.CompilerParams(
            dimension_semantics=("parallel", "arbitrary"),
            vmem_limit_bytes=_VMEM_LIMIT),
    )(x, w, b)


def _conv1_1(x_nchw, w_eff, b_eff):
    n, _, h, wd = x_nchw.shape
    xpb = jnp.pad(x_nchw, ((0, 0), (0, 0), (1, 1), (1, 1)),
                  mode="reflect").astype(jnp.bfloat16)
    pt = jnp.stack(
        [xpb[:, c, di:di + h, dj:dj + wd].reshape(-1)
         for di in range(3) for dj in range(3) for c in range(3)],
        axis=0)
    m = n * h * wd
    tm = min(16384, m)
    out = pl.pallas_call(
        _mm_t_bias_relu_kernel,
        out_shape=jax.ShapeDtypeStruct((m, 64), jnp.bfloat16),
        grid=(m // tm,),
        in_specs=[
            pl.BlockSpec((27, tm), lambda i: (0, i)),
            pl.BlockSpec((27, 64), lambda i: (0, 0)),
            pl.BlockSpec((1, 64), lambda i: (0, 0)),
        ],
        out_specs=pl.BlockSpec((tm, 64), lambda i: (i, 0)),
        compiler_params=pltpu.CompilerParams(
            dimension_semantics=("parallel",),
            vmem_limit_bytes=_VMEM_LIMIT),
    )(pt, w_eff, b_eff)
    return out.reshape(n, h, wd, 64)


def kernel(x_nchw,
           conv0_w, conv0_b,
           conv1_1_w, conv1_1_b,
           conv1_2_w, conv1_2_b,
           conv2_1_w, conv2_1_b,
           conv2_2_w, conv2_2_b,
           conv3_1_w, conv3_1_b,
           conv3_2_w, conv3_2_b,
           conv3_3_w, conv3_3_b,
           conv3_4_w, conv3_4_b,
           conv4_1_w, conv4_1_b):
    bf = jnp.bfloat16

    w1 = conv1_1_w.reshape(9, 3, 64)
    w_eff = jnp.einsum("ck,tkm->tcm", conv0_w, w1).reshape(27, 64)
    b_eff = conv1_1_b + jnp.einsum("k,tkm->m", conv0_b, w1)

    def wb(wm, bv):
        return wm.astype(bf), bv.reshape(1, -1)

    w11, b11 = wb(w_eff, b_eff)
    y = _conv1_1(x_nchw, w11, b11)

    pool1_size = (y.shape[0], 64, y.shape[1], y.shape[2])
    y, idx1 = _conv3x3_pool(y, *wb(conv1_2_w, conv1_2_b), 16)

    y = _conv3x3(y, *wb(conv2_1_w, conv2_1_b), 32)
    pool2_size = (y.shape[0], 128, y.shape[1], y.shape[2])
    y, idx2 = _conv3x3_pool(y, *wb(conv2_2_w, conv2_2_b), 32)

    y = _conv3x3(y, *wb(conv3_1_w, conv3_1_b), 64)
    y = _conv3x3(y, *wb(conv3_2_w, conv3_2_b), 32)
    y = _conv3x3(y, *wb(conv3_3_w, conv3_3_b), 32)
    pool3_size = (y.shape[0], 256, y.shape[1], y.shape[2])
    y, idx3 = _conv3x3_pool(y, *wb(conv3_4_w, conv3_4_b), 32)

    out = _conv3x3(y, *wb(conv4_1_w, conv4_1_b), 32)

    return (out, idx1, pool1_size,
            idx2, pool2_size,
            idx3, pool3_size)


# speedup vs baseline: 1.1085x; 1.1085x over previous
import functools

import jax
import jax.numpy as jnp
from jax.experimental import pallas as pl
from jax.experimental.pallas import tpu as pltpu

_VMEM_LIMIT = 57 * 1024 * 1024


def _mm_t_bias_relu_kernel(x_ref, w_ref, b_ref, o_ref):
    acc = jax.lax.dot_general(x_ref[...], w_ref[...],
                              (((0,), (0,)), ((), ())),
                              preferred_element_type=jnp.float32)
    acc = jnp.maximum(acc + b_ref[...], 0.0)
    o_ref[...] = acc.astype(o_ref.dtype)


def _conv3x3_kernel(x_ref, w_ref, b_ref, o_ref, xp_s, *, th, w_out, cin):
    h = pl.program_id(1)
    hh = x_ref.shape[1]

    @pl.when(h == 0)
    def _():
        for r in range(0, hh, 64):
            rr = min(64, hh - r)
            xp_s[1 + r:1 + r + rr, 1:w_out + 1, :] = x_ref[0, r:r + rr, :, :]
        xp_s[0:1, 1:w_out + 1, :] = x_ref[0, 1:2, :, :]
        xp_s[hh + 1:hh + 2, 1:w_out + 1, :] = x_ref[0, hh - 2:hh - 1, :, :]
        xp_s[:, 0:1, :] = xp_s[:, 2:3, :]
        xp_s[:, w_out + 1:w_out + 2, :] = xp_s[:, w_out - 1:w_out, :]

    rows = xp_s[pl.ds(h * th, th + 2), :, :]
    pieces = [rows[di:di + th, dj:dj + w_out, :]
              for di in range(3) for dj in range(3)]
    patches = jnp.concatenate(pieces, axis=-1)
    p2 = patches.reshape(th * w_out, 9 * cin)
    acc = jnp.dot(p2, w_ref[...], preferred_element_type=jnp.float32)
    acc = jnp.maximum(acc + b_ref[...], 0.0)
    o_ref[0, ...] = acc.reshape(th, w_out, -1).astype(o_ref.dtype)


def _bf16_lo(q):
    return pltpu.bitcast(q << 16, jnp.float32)


def _bf16_hi(q):
    return pltpu.bitcast(q & jnp.int32(-65536), jnp.float32)


def _argmax4(v0, v1, v2, v3, w_in):
    best = v0
    delta = jnp.zeros(best.shape, jnp.int32)
    for v, d in ((v1, 1), (v2, w_in), (v3, w_in + 1)):
        upd = v > best
        best = jnp.where(upd, v, best)
        delta = jnp.where(upd, jnp.int32(d), delta)
    return best, delta


def _conv3x3_pool_kernel(x_ref, w_ref, b_ref, o_ref, i_ref, xp_s,
                         *, th, w_out, cin):
    h = pl.program_id(1)
    hh = x_ref.shape[1]

    @pl.when(h == 0)
    def _():
        for r in range(0, hh, 64):
            rr = min(64, hh - r)
            xp_s[1 + r:1 + r + rr, 1:w_out + 1, :] = x_ref[0, r:r + rr, :, :]
        xp_s[0:1, 1:w_out + 1, :] = x_ref[0, 1:2, :, :]
        xp_s[hh + 1:hh + 2, 1:w_out + 1, :] = x_ref[0, hh - 2:hh - 1, :, :]
        xp_s[:, 0:1, :] = xp_s[:, 2:3, :]
        xp_s[:, w_out + 1:w_out + 2, :] = xp_s[:, w_out - 1:w_out, :]

    rows = xp_s[pl.ds(h * th, th + 2), :, :]
    pieces = [rows[di:di + th, dj:dj + w_out, :]
              for di in range(3) for dj in range(3)]
    patches = jnp.concatenate(pieces, axis=-1)
    p2 = patches.reshape(th * w_out, 9 * cin)
    acc = jnp.dot(p2, w_ref[...], preferred_element_type=jnp.float32)
    acc = jnp.maximum(acc + b_ref[...], 0.0)
    cout = w_ref.shape[1]
    y = acc.astype(jnp.bfloat16).reshape(th, w_out, cout)
    zi = pltpu.bitcast(y, jnp.int32)
    z = zi.reshape(th // 2, 2, w_out // 2, cout)
    r0, r1 = z[:, 0], z[:, 1]
    best, delta = _argmax4(_bf16_lo(r0), _bf16_hi(r0),
                           _bf16_lo(r1), _bf16_hi(r1), w_out)
    hrow = h * (th // 2) + jax.lax.broadcasted_iota(jnp.int32, best.shape, 0)
    ww = jax.lax.broadcasted_iota(jnp.int32, best.shape, 1)
    o_ref[0, ...] = best.astype(o_ref.dtype)
    i_ref[0, ...] = 2 * hrow * w_out + 2 * ww + delta


def _conv3x3(x, w, b, th_pref):
    n, h, wd, c = x.shape
    cout = w.shape[1]
    th = min(th_pref, h)
    out_dtype = jnp.float32 if cout == 512 else jnp.bfloat16
    out_spec = pl.BlockSpec((1, th, wd, cout), lambda i, j: (i, j, 0, 0))
    return pl.pallas_call(
        functools.partial(_conv3x3_kernel, th=th, w_out=wd, cin=c),
        out_shape=jax.ShapeDtypeStruct((n, h, wd, cout), out_dtype),
        grid=(n, h // th),
        in_specs=[
            pl.BlockSpec((1, h, wd, c), lambda i, j: (i, 0, 0, 0)),
            pl.BlockSpec((9 * c, cout), lambda i, j: (0, 0)),
            pl.BlockSpec((1, cout), lambda i, j: (0, 0)),
        ],
        out_specs=out_spec,
        scratch_shapes=[pltpu.VMEM((h + 2, wd + 2, c), jnp.bfloat16)],
        compiler_params=pltpu.CompilerParams(
            dimension_semantics=("parallel", "arbitrary"),
            vmem_limit_bytes=_VMEM_LIMIT),
    )(x, w, b)


def _conv3x3_pool(x, w, b, th_pref):
    n, h, wd, c = x.shape
    cout = w.shape[1]
    th = min(th_pref, h)
    return pl.pallas_call(
        functools.partial(_conv3x3_pool_kernel, th=th, w_out=wd, cin=c),
        out_shape=(
            jax.ShapeDtypeStruct((n, h // 2, wd // 2, cout), jnp.bfloat16),
            jax.ShapeDtypeStruct((n, h // 2, wd // 2, cout), jnp.int32)),
        grid=(n, h // th),
        in_specs=[
            pl.BlockSpec((1, h, wd, c), lambda i, j: (i, 0, 0, 0)),
            pl.BlockSpec((9 * c, cout), lambda i, j: (0, 0)),
            pl.BlockSpec((1, cout), lambda i, j: (0, 0)),
        ],
        out_specs=(
            pl.BlockSpec((1, th // 2, wd // 2, cout),
                         lambda i, j: (i, j, 0, 0)),
            pl.BlockSpec((1, th // 2, wd // 2, cout),
                         lambda i, j: (i, j, 0, 0))),
        scratch_shapes=[pltpu.VMEM((h + 2, wd + 2, c), jnp.bfloat16)],
        compiler_params=pltpu.CompilerParams(
            dimension_semantics=("parallel", "arbitrary"),
            vmem_limit_bytes=_VMEM_LIMIT),
    )(x, w, b)


def _conv1_1(x_nchw, w_eff, b_eff):
    n, _, h, wd = x_nchw.shape
    xpb = jnp.pad(x_nchw, ((0, 0), (0, 0), (1, 1), (1, 1)),
                  mode="reflect").astype(jnp.bfloat16)
    pt = jnp.stack(
        [xpb[:, c, di:di + h, dj:dj + wd].reshape(-1)
         for di in range(3) for dj in range(3) for c in range(3)],
        axis=0)
    m = n * h * wd
    tm = min(16384, m)
    out = pl.pallas_call(
        _mm_t_bias_relu_kernel,
        out_shape=jax.ShapeDtypeStruct((m, 64), jnp.bfloat16),
        grid=(m // tm,),
        in_specs=[
            pl.BlockSpec((27, tm), lambda i: (0, i)),
            pl.BlockSpec((27, 64), lambda i: (0, 0)),
            pl.BlockSpec((1, 64), lambda i: (0, 0)),
        ],
        out_specs=pl.BlockSpec((tm, 64), lambda i: (i, 0)),
        compiler_params=pltpu.CompilerParams(
            dimension_semantics=("parallel",),
            vmem_limit_bytes=_VMEM_LIMIT),
    )(pt, w_eff, b_eff)
    return out.reshape(n, h, wd, 64)


def kernel(x_nchw,
           conv0_w, conv0_b,
           conv1_1_w, conv1_1_b,
           conv1_2_w, conv1_2_b,
           conv2_1_w, conv2_1_b,
           conv2_2_w, conv2_2_b,
           conv3_1_w, conv3_1_b,
           conv3_2_w, conv3_2_b,
           conv3_3_w, conv3_3_b,
           conv3_4_w, conv3_4_b,
           conv4_1_w, conv4_1_b):
    bf = jnp.bfloat16

    w1 = conv1_1_w.reshape(9, 3, 64)
    w_eff = jnp.einsum("ck,tkm->tcm", conv0_w, w1).reshape(27, 64)
    b_eff = conv1_1_b + jnp.einsum("k,tkm->m", conv0_b, w1)

    def wb(wm, bv):
        return wm.astype(bf), bv.reshape(1, -1)

    w11, b11 = wb(w_eff, b_eff)
    y = _conv1_1(x_nchw, w11, b11)

    pool1_size = (y.shape[0], 64, y.shape[1], y.shape[2])
    y, idx1 = _conv3x3_pool(y, *wb(conv1_2_w, conv1_2_b), 16)

    y = _conv3x3(y, *wb(conv2_1_w, conv2_1_b), 32)
    pool2_size = (y.shape[0], 128, y.shape[1], y.shape[2])
    y, idx2 = _conv3x3_pool(y, *wb(conv2_2_w, conv2_2_b), 32)

    y = _conv3x3(y, *wb(conv3_1_w, conv3_1_b), 64)
    y = _conv3x3(y, *wb(conv3_2_w, conv3_2_b), 32)
    y = _conv3x3(y, *wb(conv3_3_w, conv3_3_b), 32)
    pool3_size = (y.shape[0], 256, y.shape[1], y.shape[2])
    y, idx3 = _conv3x3_pool(y, *wb(conv3_4_w, conv3_4_b), 32)

    out = _conv3x3(y, *wb(conv4_1_w, conv4_1_b), 32)

    def to_nchw(a):
        return jnp.transpose(a, (0, 3, 1, 2))

    return (to_nchw(out), to_nchw(idx1), pool1_size,
            to_nchw(idx2), pool2_size,
            to_nchw(idx3), pool3_size)


# speedup vs baseline: 1.3267x; 1.1969x over previous
import functools

import jax
import jax.numpy as jnp
from jax.experimental import pallas as pl
from jax.experimental.pallas import tpu as pltpu

_VMEM_LIMIT = 57 * 1024 * 1024


def _mm_bias_relu_kernel(x_ref, w_ref, b_ref, o_ref):
    acc = jnp.dot(x_ref[...], w_ref[...], preferred_element_type=jnp.float32)
    acc = jnp.maximum(acc + b_ref[...], 0.0)
    o_ref[...] = acc.astype(o_ref.dtype)


def _conv3x3_kernel(x_ref, w_ref, b_ref, o_ref, xp_s, *, th, w_out, cin):
    h = pl.program_id(1)
    hh = x_ref.shape[1]

    @pl.when(h == 0)
    def _():
        for r in range(0, hh, 64):
            rr = min(64, hh - r)
            xp_s[1 + r:1 + r + rr, 1:w_out + 1, :] = x_ref[0, r:r + rr, :, :]
        xp_s[0:1, 1:w_out + 1, :] = x_ref[0, 1:2, :, :]
        xp_s[hh + 1:hh + 2, 1:w_out + 1, :] = x_ref[0, hh - 2:hh - 1, :, :]
        xp_s[:, 0:1, :] = xp_s[:, 2:3, :]
        xp_s[:, w_out + 1:w_out + 2, :] = xp_s[:, w_out - 1:w_out, :]

    rows = xp_s[pl.ds(h * th, th + 2), :, :]
    pieces = [rows[di:di + th, dj:dj + w_out, :]
              for di in range(3) for dj in range(3)]
    patches = jnp.concatenate(pieces, axis=-1)
    p2 = patches.reshape(th * w_out, 9 * cin)
    acc = jnp.dot(p2, w_ref[...], preferred_element_type=jnp.float32)
    acc = jnp.maximum(acc + b_ref[...], 0.0)
    o_ref[0, ...] = acc.reshape(th, w_out, -1).astype(o_ref.dtype)


def _bf16_lo(q):
    return pltpu.bitcast(q << 16, jnp.float32)


def _bf16_hi(q):
    return pltpu.bitcast(q & jnp.int32(-65536), jnp.float32)


def _argmax4(v0, v1, v2, v3, w_in):
    best = v0
    delta = jnp.zeros(best.shape, jnp.int32)
    for v, d in ((v1, 1), (v2, w_in), (v3, w_in + 1)):
        upd = v > best
        best = jnp.where(upd, v, best)
        delta = jnp.where(upd, jnp.int32(d), delta)
    return best, delta


def _conv3x3_pool_kernel(x_ref, w_ref, b_ref, o_ref, i_ref, xp_s,
                         *, th, w_out, cin):
    h = pl.program_id(1)
    hh = x_ref.shape[1]

    @pl.when(h == 0)
    def _():
        for r in range(0, hh, 64):
            rr = min(64, hh - r)
            xp_s[1 + r:1 + r + rr, 1:w_out + 1, :] = x_ref[0, r:r + rr, :, :]
        xp_s[0:1, 1:w_out + 1, :] = x_ref[0, 1:2, :, :]
        xp_s[hh + 1:hh + 2, 1:w_out + 1, :] = x_ref[0, hh - 2:hh - 1, :, :]
        xp_s[:, 0:1, :] = xp_s[:, 2:3, :]
        xp_s[:, w_out + 1:w_out + 2, :] = xp_s[:, w_out - 1:w_out, :]

    rows = xp_s[pl.ds(h * th, th + 2), :, :]
    pieces = [rows[di:di + th, dj:dj + w_out, :]
              for di in range(3) for dj in range(3)]
    patches = jnp.concatenate(pieces, axis=-1)
    p2 = patches.reshape(th * w_out, 9 * cin)
    acc = jnp.dot(p2, w_ref[...], preferred_element_type=jnp.float32)
    acc = jnp.maximum(acc + b_ref[...], 0.0)
    cout = w_ref.shape[1]
    y = acc.astype(jnp.bfloat16).reshape(th, w_out, cout)
    zi = pltpu.bitcast(y, jnp.int32)
    z = zi.reshape(th // 2, 2, w_out // 2, cout)
    r0, r1 = z[:, 0], z[:, 1]
    best, delta = _argmax4(_bf16_lo(r0), _bf16_hi(r0),
                           _bf16_lo(r1), _bf16_hi(r1), w_out)
    hrow = h * (th // 2) + jax.lax.broadcasted_iota(jnp.int32, best.shape, 0)
    ww = jax.lax.broadcasted_iota(jnp.int32, best.shape, 1)
    o_ref[0, ...] = best.astype(o_ref.dtype)
    i_ref[0, ...] = 2 * hrow * w_out + 2 * ww + delta


def _conv3x3(x, w, b, th_pref):
    n, h, wd, c = x.shape
    cout = w.shape[1]
    th = min(th_pref, h)
    out_dtype = jnp.float32 if cout == 512 else jnp.bfloat16
    out_spec = pl.BlockSpec((1, th, wd, cout), lambda i, j: (i, j, 0, 0))
    return pl.pallas_call(
        functools.partial(_conv3x3_kernel, th=th, w_out=wd, cin=c),
        out_shape=jax.ShapeDtypeStruct((n, h, wd, cout), out_dtype),
        grid=(n, h // th),
        in_specs=[
            pl.BlockSpec((1, h, wd, c), lambda i, j: (i, 0, 0, 0)),
            pl.BlockSpec((9 * c, cout), lambda i, j: (0, 0)),
            pl.BlockSpec((1, cout), lambda i, j: (0, 0)),
        ],
        out_specs=out_spec,
        scratch_shapes=[pltpu.VMEM((h + 2, wd + 2, c), jnp.bfloat16)],
        compiler_params=pltpu.CompilerParams(
            dimension_semantics=("parallel", "arbitrary"),
            vmem_limit_bytes=_VMEM_LIMIT),
    )(x, w, b)


def _conv3x3_pool(x, w, b, th_pref):
    n, h, wd, c = x.shape
    cout = w.shape[1]
    th = min(th_pref, h)
    return pl.pallas_call(
        functools.partial(_conv3x3_pool_kernel, th=th, w_out=wd, cin=c),
        out_shape=(
            jax.ShapeDtypeStruct((n, h // 2, wd // 2, cout), jnp.bfloat16),
            jax.ShapeDtypeStruct((n, h // 2, wd // 2, cout), jnp.int32)),
        grid=(n, h // th),
        in_specs=[
            pl.BlockSpec((1, h, wd, c), lambda i, j: (i, 0, 0, 0)),
            pl.BlockSpec((9 * c, cout), lambda i, j: (0, 0)),
            pl.BlockSpec((1, cout), lambda i, j: (0, 0)),
        ],
        out_specs=(
            pl.BlockSpec((1, th // 2, wd // 2, cout),
                         lambda i, j: (i, j, 0, 0)),
            pl.BlockSpec((1, th // 2, wd // 2, cout),
                         lambda i, j: (i, j, 0, 0))),
        scratch_shapes=[pltpu.VMEM((h + 2, wd + 2, c), jnp.bfloat16)],
        compiler_params=pltpu.CompilerParams(
            dimension_semantics=("parallel", "arbitrary"),
            vmem_limit_bytes=_VMEM_LIMIT),
    )(x, w, b)


def _conv1_1(x_nhwc_bf16, w_eff, b_eff):
    n, h, wd, _ = x_nhwc_bf16.shape
    xp = jnp.pad(x_nhwc_bf16, ((0, 0), (1, 1), (1, 1), (0, 0)),
                 mode="reflect")
    patches = jnp.concatenate(
        [xp[:, di:di + h, dj:dj + wd, :]
         for di in range(3) for dj in range(3)],
        axis=-1).reshape(n * h * wd, 27)
    m = n * h * wd
    tm = min(8192, m)
    out = pl.pallas_call(
        _mm_bias_relu_kernel,
        out_shape=jax.ShapeDtypeStruct((m, 64), jnp.bfloat16),
        grid=(m // tm,),
        in_specs=[
            pl.BlockSpec((tm, 27), lambda i: (i, 0)),
            pl.BlockSpec((27, 64), lambda i: (0, 0)),
            pl.BlockSpec((1, 64), lambda i: (0, 0)),
        ],
        out_specs=pl.BlockSpec((tm, 64), lambda i: (i, 0)),
        compiler_params=pltpu.CompilerParams(
            dimension_semantics=("parallel",),
            vmem_limit_bytes=_VMEM_LIMIT),
    )(patches, w_eff, b_eff)
    return out.reshape(n, h, wd, 64)


def kernel(x_nchw,
           conv0_w, conv0_b,
           conv1_1_w, conv1_1_b,
           conv1_2_w, conv1_2_b,
           conv2_1_w, conv2_1_b,
           conv2_2_w, conv2_2_b,
           conv3_1_w, conv3_1_b,
           conv3_2_w, conv3_2_b,
           conv3_3_w, conv3_3_b,
           conv3_4_w, conv3_4_b,
           conv4_1_w, conv4_1_b):
    bf = jnp.bfloat16
    x = jnp.transpose(x_nchw, (0, 2, 3, 1)).astype(bf)

    w1 = conv1_1_w.reshape(9, 3, 64)
    w_eff = jnp.einsum("ck,tkm->tcm", conv0_w, w1).reshape(27, 64)
    b_eff = conv1_1_b + jnp.einsum("k,tkm->m", conv0_b, w1)

    def wb(wm, bv):
        return wm.astype(bf), bv.reshape(1, -1)

    w11, b11 = wb(w_eff, b_eff)
    y = _conv1_1(x, w11, b11)

    pool1_size = (y.shape[0], 64, y.shape[1], y.shape[2])
    y, idx1 = _conv3x3_pool(y, *wb(conv1_2_w, conv1_2_b), 16)

    y = _conv3x3(y, *wb(conv2_1_w, conv2_1_b), 32)
    pool2_size = (y.shape[0], 128, y.shape[1], y.shape[2])
    y, idx2 = _conv3x3_pool(y, *wb(conv2_2_w, conv2_2_b), 32)

    y = _conv3x3(y, *wb(conv3_1_w, conv3_1_b), 64)
    y = _conv3x3(y, *wb(conv3_2_w, conv3_2_b), 32)
    y = _conv3x3(y, *wb(conv3_3_w, conv3_3_b), 32)
    pool3_size = (y.shape[0], 256, y.shape[1], y.shape[2])
    y, idx3 = _conv3x3_pool(y, *wb(conv3_4_w, conv3_4_b), 32)

    out = _conv3x3(y, *wb(conv4_1_w, conv4_1_b), 32)

    def to_nchw(a):
        return jnp.transpose(a, (0, 3, 1, 2))

    return (to_nchw(out), to_nchw(idx1), pool1_size,
            to_nchw(idx2), pool2_size,
            to_nchw(idx3), pool3_size)
